# baseline (speedup 1.0000x reference)
import functools

import jax
import jax.numpy as jnp
from jax import lax
from jax.experimental import pallas as pl
from jax.experimental.pallas import tpu as pltpu
from jax.experimental.pallas import tpu_sc as plsc

_BBB = 2048
_CH = 640
_DP = 128
_NSPLIT = 4


def _argmax_body(x_ref, out_ref):
    x = x_ref[...]
    n = x.shape[1]
    m = jnp.max(x, axis=1, keepdims=True)
    ii = lax.broadcasted_iota(jnp.int32, x.shape, 1)
    cand = jnp.where(x == m, ii, n)
    out_ref[...] = jnp.min(cand, axis=1)[:, None, :]


def _tc_argmax_part(xp, c, sc):
    S, N, B = xp.shape
    return pl.pallas_call(
        _argmax_body,
        grid=(sc, B // _BBB),
        in_specs=[pl.BlockSpec((1, N, _BBB), lambda s, i: (c * sc + s, 0, i))],
        out_specs=pl.BlockSpec((1, 1, _BBB), lambda s, i: (s, 0, i)),
        out_shape=jax.ShapeDtypeStruct((sc, 1, B), jnp.int32),
    )(xp)


@functools.lru_cache(maxsize=None)
def _make_sc_gather(total):
    info = plsc.get_sparse_core_info()
    NC, NS = info.num_cores, info.num_subcores
    per_w = total // (NC * NS)
    mesh = plsc.VectorSubcoreMesh(core_axis_name="c", subcore_axis_name="s")

    @functools.partial(
        pl.kernel,
        out_type=jax.ShapeDtypeStruct((total, _DP), jnp.float32),
        mesh=mesh,
        scratch_types=[
            pltpu.VMEM((_CH,), jnp.int32),
            pltpu.VMEM((_CH, _DP), jnp.float32),
            pltpu.SemaphoreType.DMA,
        ],
    )
    def gather(tab_hbm, idx_hbm, out_hbm, idx_v, rows_v, sem):
        wid = lax.axis_index("s") * NC + lax.axis_index("c")
        base = wid * per_w
        for k in range(per_w // _CH):
            off = base + k * _CH
            pltpu.sync_copy(idx_hbm.at[pl.ds(off, _CH)], idx_v)
            pltpu.async_copy(tab_hbm.at[idx_v], rows_v, sem).wait()
            pltpu.sync_copy(rows_v, out_hbm.at[pl.ds(off, _CH)])

    return gather


def kernel(x, state_embedding):
    B, S, N = x.shape
    D = state_embedding.shape[1]
    xp = jnp.transpose(x, (1, 2, 0))
    tabp = jnp.pad(state_embedding, ((0, 0), (0, _DP - D)))
    sc = S // _NSPLIT
    parts = []
    for c in range(_NSPLIT):
        idx = _tc_argmax_part(xp, c, sc)
        outp = _make_sc_gather(sc * B)(tabp, idx.reshape(sc * B))
        parts.append(outp[:, :D].reshape(sc, B, D))
    return jnp.concatenate(parts, axis=0)


# speedup vs baseline: 1.1237x; 1.1237x over previous
import functools

import jax
import jax.numpy as jnp
from jax import lax
from jax.experimental import pallas as pl
from jax.experimental.pallas import tpu as pltpu
from jax.experimental.pallas import tpu_sc as plsc

_BBB = 2048
_CH = 640
_DP = 128
_NSPLIT = 1


def _argmax_body(x_ref, out_ref):
    x = x_ref[...]
    n = x.shape[1]
    m = jnp.max(x, axis=1, keepdims=True)
    ii = lax.broadcasted_iota(jnp.int32, x.shape, 1)
    cand = jnp.where(x == m, ii, n)
    out_ref[...] = jnp.min(cand, axis=1)[:, None, :]


def _tc_argmax_part(xp, c, sc):
    S, N, B = xp.shape
    return pl.pallas_call(
        _argmax_body,
        grid=(sc, B // _BBB),
        in_specs=[pl.BlockSpec((1, N, _BBB), lambda s, i: (c * sc + s, 0, i))],
        out_specs=pl.BlockSpec((1, 1, _BBB), lambda s, i: (s, 0, i)),
        out_shape=jax.ShapeDtypeStruct((sc, 1, B), jnp.int32),
    )(xp)


@functools.lru_cache(maxsize=None)
def _make_sc_gather(total):
    info = plsc.get_sparse_core_info()
    NC, NS = info.num_cores, info.num_subcores
    per_w = total // (NC * NS)
    mesh = plsc.VectorSubcoreMesh(core_axis_name="c", subcore_axis_name="s")

    @functools.partial(
        pl.kernel,
        out_type=jax.ShapeDtypeStruct((total, _DP), jnp.float32),
        mesh=mesh,
        scratch_types=[
            pltpu.VMEM((_CH,), jnp.int32),
            pltpu.VMEM((_CH, _DP), jnp.float32),
            pltpu.SemaphoreType.DMA,
        ],
    )
    def gather(tab_hbm, idx_hbm, out_hbm, idx_v, rows_v, sem):
        wid = lax.axis_index("s") * NC + lax.axis_index("c")
        base = wid * per_w
        for k in range(per_w // _CH):
            off = base + k * _CH
            pltpu.sync_copy(idx_hbm.at[pl.ds(off, _CH)], idx_v)
            pltpu.async_copy(tab_hbm.at[idx_v], rows_v, sem).wait()
            pltpu.sync_copy(rows_v, out_hbm.at[pl.ds(off, _CH)])

    return gather


def kernel(x, state_embedding):
    B, S, N = x.shape
    D = state_embedding.shape[1]
    xp = jnp.transpose(x, (1, 2, 0))
    tabp = jnp.pad(state_embedding, ((0, 0), (0, _DP - D)))
    sc = S // _NSPLIT
    parts = []
    for c in range(_NSPLIT):
        idx = _tc_argmax_part(xp, c, sc)
        outp = _make_sc_gather(sc * B)(tabp, idx.reshape(sc * B))
        parts.append(outp[:, :D].reshape(sc, B, D))
    return jnp.concatenate(parts, axis=0)


# speedup vs baseline: 1.1497x; 1.0231x over previous
import functools

import jax
import jax.numpy as jnp
from jax import lax
from jax.experimental import pallas as pl
from jax.experimental.pallas import tpu as pltpu
from jax.experimental.pallas import tpu_sc as plsc

_BBB = 2048
_CH = 640
_DP = 128
_NSPLIT = 1


def _argmax_body(x_ref, out_ref):
    n = x_ref.shape[1]
    B = x_ref.shape[2]
    nslab = n // 8

    def step(k, carry):
        m8, k8 = carry
        slab = x_ref[0, pl.ds(k * 8, 8), :]
        gt = slab > m8
        return jnp.where(gt, slab, m8), jnp.where(gt, k, k8)

    m0 = x_ref[0, pl.ds(0, 8), :]
    k0 = jnp.zeros((8, B), jnp.int32)
    m8, k8 = lax.fori_loop(1, nslab, step, (m0, k0), unroll=4)

    idx8 = k8 * 8 + lax.broadcasted_iota(jnp.int32, (8, B), 0)
    mf = jnp.max(m8, axis=0, keepdims=True)
    cand = jnp.where(m8 == mf, idx8, n)
    out_ref[...] = jnp.min(cand, axis=0)[None, None, :]


def _tc_argmax_part(xp, c, sc):
    S, N, B = xp.shape
    return pl.pallas_call(
        _argmax_body,
        grid=(sc, B // _BBB),
        in_specs=[pl.BlockSpec((1, N, _BBB), lambda s, i: (c * sc + s, 0, i))],
        out_specs=pl.BlockSpec((1, 1, _BBB), lambda s, i: (s, 0, i)),
        out_shape=jax.ShapeDtypeStruct((sc, 1, B), jnp.int32),
    )(xp)


@functools.lru_cache(maxsize=None)
def _make_sc_gather(total):
    info = plsc.get_sparse_core_info()
    NC, NS = info.num_cores, info.num_subcores
    per_w = total // (NC * NS)
    mesh = plsc.VectorSubcoreMesh(core_axis_name="c", subcore_axis_name="s")

    @functools.partial(
        pl.kernel,
        out_type=jax.ShapeDtypeStruct((total, _DP), jnp.float32),
        mesh=mesh,
        scratch_types=[
            pltpu.VMEM((_CH,), jnp.int32),
            pltpu.VMEM((_CH, _DP), jnp.float32),
            pltpu.SemaphoreType.DMA,
        ],
    )
    def gather(tab_hbm, idx_hbm, out_hbm, idx_v, rows_v, sem):
        wid = lax.axis_index("s") * NC + lax.axis_index("c")
        base = wid * per_w
        for k in range(per_w // _CH):
            off = base + k * _CH
            pltpu.sync_copy(idx_hbm.at[pl.ds(off, _CH)], idx_v)
            pltpu.async_copy(tab_hbm.at[idx_v], rows_v, sem).wait()
            pltpu.sync_copy(rows_v, out_hbm.at[pl.ds(off, _CH)])

    return gather


def kernel(x, state_embedding):
    B, S, N = x.shape
    D = state_embedding.shape[1]
    xp = jnp.transpose(x, (1, 2, 0))
    tabp = jnp.pad(state_embedding, ((0, 0), (0, _DP - D)))
    sc = S // _NSPLIT
    parts = []
    for c in range(_NSPLIT):
        idx = _tc_argmax_part(xp, c, sc)
        outp = _make_sc_gather(sc * B)(tabp, idx.reshape(sc * B))
        parts.append(outp[:, :D].reshape(sc, B, D))
    return jnp.concatenate(parts, axis=0)
